# baseline (device time: 852513 ns/iter reference)
import jax
import jax.numpy as jnp
from jax import lax
from jax.experimental import pallas as pl
from jax.experimental.pallas import tpu as pltpu

N_DEV = 16
N_EXPERTS = 64
CAPACITY = 409
L = 128


def kernel(x, router_W, route_idx, expert_W):
    n_tok, d_model = x.shape
    e_per, _, d_out = expert_W.shape

    e = route_idx[:, 0]
    oh = jax.nn.one_hot(e, N_EXPERTS, dtype=jnp.int32)
    hist = jnp.sum(oh, axis=0, dtype=jnp.int32)
    local_rank = jnp.cumsum(oh, axis=0) - oh
    local_rank = jnp.take_along_axis(local_rank, e[:, None], axis=1)[:, 0]

    P = jnp.zeros((N_EXPERTS, L, d_model), jnp.bfloat16)
    P = P.at[e, local_rank].set(x.astype(jnp.bfloat16), mode="drop")
    W = expert_W.astype(jnp.bfloat16)

    def body(P_ref, W_ref, hist_ref, out_ref, hists_ref, wbuf, hbuf,
             wsend, wrecv, hsend, hrecv):
        my = lax.axis_index("i")
        left = (my + N_DEV - 1) % N_DEV
        right = (my + 1) % N_DEV

        barrier = pltpu.get_barrier_semaphore()
        for nbr in (left, right):
            pl.semaphore_signal(barrier, inc=1, device_id=(nbr,),
                                device_id_type=pl.DeviceIdType.MESH)
        pl.semaphore_wait(barrier, 2)

        wbuf[0] = W_ref[...]
        hbuf[0] = hist_ref[...]
        hists_ref[pl.ds(my, 1)] = hist_ref[...][None]

        def mm(idx, w):
            a = P_ref[pl.ds(idx, 1)][0]
            r = lax.dot_general(a, w, (((1,), (0,)), ((), ())),
                                preferred_element_type=jnp.float32)
            out_ref[pl.ds(idx, 1)] = r.astype(jnp.bfloat16)[None]

        for k in range(e_per):
            mm(my * e_per + k, W_ref[k])

        for h in range(N_DEV - 1):
            ss, rs = h % 2, (h + 1) % 2
            wr = pltpu.make_async_remote_copy(
                src_ref=wbuf.at[ss], dst_ref=wbuf.at[rs],
                send_sem=wsend.at[ss], recv_sem=wrecv.at[rs],
                device_id=(right,), device_id_type=pl.DeviceIdType.MESH)
            hr = pltpu.make_async_remote_copy(
                src_ref=hbuf.at[ss], dst_ref=hbuf.at[rs],
                send_sem=hsend.at[ss], recv_sem=hrecv.at[rs],
                device_id=(right,), device_id_type=pl.DeviceIdType.MESH)
            wr.start()
            hr.start()
            wr.wait()
            hr.wait()
            src = (my + (N_DEV - 1 - h)) % N_DEV
            hists_ref[pl.ds(src, 1)] = hbuf[rs][None]
            for k in range(e_per):
                mm(src * e_per + k, wbuf[rs, k])

    out_P, hists = pl.pallas_call(
        body,
        out_shape=[
            jax.ShapeDtypeStruct((N_EXPERTS, L, d_out), jnp.bfloat16),
            jax.ShapeDtypeStruct((N_DEV, 1, N_EXPERTS), jnp.int32),
        ],
        in_specs=[pl.BlockSpec(memory_space=pltpu.VMEM)] * 3,
        out_specs=[pl.BlockSpec(memory_space=pltpu.VMEM)] * 2,
        scratch_shapes=[
            pltpu.VMEM((2, e_per, d_model, d_out), jnp.bfloat16),
            pltpu.VMEM((2, 1, N_EXPERTS), jnp.int32),
            pltpu.SemaphoreType.DMA((2,)),
            pltpu.SemaphoreType.DMA((2,)),
            pltpu.SemaphoreType.DMA((2,)),
            pltpu.SemaphoreType.DMA((2,)),
        ],
        compiler_params=pltpu.CompilerParams(collective_id=0),
    )(P, W, hist[None, :])

    hists = hists[:, 0, :]
    my = lax.axis_index("i")
    prior = (jnp.arange(N_DEV) < my).astype(jnp.int32)[:, None]
    gprefix = jnp.sum(hists * prior, axis=0)
    grank = gprefix[e] + local_rank
    keep = grank < CAPACITY
    out = out_P[e, local_rank]
    return jnp.where(keep[:, None], out.astype(jnp.float32), 0.0)


# device time: 547414 ns/iter; 1.5573x vs baseline; 1.5573x over previous
import jax
import jax.numpy as jnp
from jax import lax
from jax.experimental import pallas as pl
from jax.experimental.pallas import tpu as pltpu

N_DEV = 16
N_EXPERTS = 64
CAPACITY = 409
L = 128


def kernel(x, router_W, route_idx, expert_W):
    n_tok, d_model = x.shape
    e_per, _, d_out = expert_W.shape

    e = route_idx[:, 0]
    oh = jax.nn.one_hot(e, N_EXPERTS, dtype=jnp.int32)
    hist = jnp.sum(oh, axis=0, dtype=jnp.int32)
    local_rank = jnp.cumsum(oh, axis=0) - oh
    local_rank = jnp.take_along_axis(local_rank, e[:, None], axis=1)[:, 0]

    P = jnp.zeros((N_EXPERTS, L, d_model), jnp.bfloat16)
    P = P.at[e, local_rank].set(x.astype(jnp.bfloat16), mode="drop")
    W = expert_W.astype(jnp.bfloat16)

    FWD = N_DEV // 2
    BWD = N_DEV - 1 - FWD

    def body(P_ref, W_ref, hist_ref, out_ref, hists_ref,
             wbufF, wbufB, hbufF, hbufB,
             wsF, wrF, wsB, wrB, hsF, hrF, hsB, hrB):
        my = lax.axis_index("i")
        left = (my + N_DEV - 1) % N_DEV
        right = (my + 1) % N_DEV

        barrier = pltpu.get_barrier_semaphore()
        for nbr in (left, right):
            pl.semaphore_signal(barrier, inc=1, device_id=(nbr,),
                                device_id_type=pl.DeviceIdType.MESH)
        pl.semaphore_wait(barrier, 2)

        wbufF[0] = W_ref[...]
        wbufB[0] = W_ref[...]
        hbufF[0] = hist_ref[...]
        hbufB[0] = hist_ref[...]
        hists_ref[pl.ds(my, 1)] = hist_ref[...][None]

        def mm(idx, w):
            a = P_ref[pl.ds(idx, 1)][0]
            r = lax.dot_general(a, w, (((1,), (0,)), ((), ())),
                                preferred_element_type=jnp.float32)
            out_ref[pl.ds(idx, 1)] = r.astype(jnp.bfloat16)[None]

        for k in range(e_per):
            mm(my * e_per + k, W_ref[k])

        def hop(h, wbuf, hbuf, wsend, wrecv, hsend, hrecv, dst):
            ss, rs = h % 2, (h + 1) % 2
            wr_ = pltpu.make_async_remote_copy(
                src_ref=wbuf.at[ss], dst_ref=wbuf.at[rs],
                send_sem=wsend.at[ss], recv_sem=wrecv.at[rs],
                device_id=(dst,), device_id_type=pl.DeviceIdType.MESH)
            hr_ = pltpu.make_async_remote_copy(
                src_ref=hbuf.at[ss], dst_ref=hbuf.at[rs],
                send_sem=hsend.at[ss], recv_sem=hrecv.at[rs],
                device_id=(dst,), device_id_type=pl.DeviceIdType.MESH)
            wr_.start()
            hr_.start()
            return wr_, hr_

        def consume(h, wbuf, hbuf, src):
            rs = (h + 1) % 2
            hists_ref[pl.ds(src, 1)] = hbuf[rs][None]
            for k in range(e_per):
                mm(src * e_per + k, wbuf[rs, k])

        for h in range(FWD):
            f = hop(h, wbufF, hbufF, wsF, wrF, hsF, hrF, right)
            b = hop(h, wbufB, hbufB, wsB, wrB, hsB, hrB, left) if h < BWD else None
            for r_ in f:
                r_.wait()
            if b is not None:
                for r_ in b:
                    r_.wait()
            consume(h, wbufF, hbufF, (my + N_DEV - 1 - h) % N_DEV)
            if h < BWD:
                consume(h, wbufB, hbufB, (my + 1 + h) % N_DEV)

    out_P, hists = pl.pallas_call(
        body,
        out_shape=[
            jax.ShapeDtypeStruct((N_EXPERTS, L, d_out), jnp.bfloat16),
            jax.ShapeDtypeStruct((N_DEV, 1, N_EXPERTS), jnp.int32),
        ],
        in_specs=[pl.BlockSpec(memory_space=pltpu.VMEM)] * 3,
        out_specs=[pl.BlockSpec(memory_space=pltpu.VMEM)] * 2,
        scratch_shapes=[
            pltpu.VMEM((2, e_per, d_model, d_out), jnp.bfloat16),
            pltpu.VMEM((2, e_per, d_model, d_out), jnp.bfloat16),
            pltpu.VMEM((2, 1, N_EXPERTS), jnp.int32),
            pltpu.VMEM((2, 1, N_EXPERTS), jnp.int32),
        ] + [pltpu.SemaphoreType.DMA((2,))] * 8,
        compiler_params=pltpu.CompilerParams(
            collective_id=0, vmem_limit_bytes=100 * 1024 * 1024),
    )(P, W, hist[None, :])

    hists = hists[:, 0, :]
    my = lax.axis_index("i")
    prior = (jnp.arange(N_DEV) < my).astype(jnp.int32)[:, None]
    gprefix = jnp.sum(hists * prior, axis=0)
    grank = gprefix[e] + local_rank
    keep = grank < CAPACITY
    out = out_P[e, local_rank]
    return jnp.where(keep[:, None], out.astype(jnp.float32), 0.0)


# device time: 500206 ns/iter; 1.7043x vs baseline; 1.0944x over previous
import jax
import jax.numpy as jnp
from jax import lax
from jax.experimental import pallas as pl
from jax.experimental.pallas import tpu as pltpu

N_DEV = 16
N_EXPERTS = 64
CAPACITY = 409
L = 128


def kernel(x, router_W, route_idx, expert_W):
    n_tok, d_model = x.shape
    e_per, _, d_out = expert_W.shape

    e = route_idx[:, 0]
    oh = jax.nn.one_hot(e, N_EXPERTS, dtype=jnp.int32)
    hist = jnp.sum(oh, axis=0, dtype=jnp.int32)
    local_rank = jnp.cumsum(oh, axis=0) - oh
    local_rank = jnp.take_along_axis(local_rank, e[:, None], axis=1)[:, 0]

    xb = x.astype(jnp.bfloat16)
    W = expert_W.astype(jnp.bfloat16)

    FWD = N_DEV // 2
    BWD = N_DEV - 1 - FWD
    R = e_per * L

    def body(x_ref, W_ref, hist_ref, e_ref, rank_ref, out_ref, hists_ref,
             wbufF, wbufB, hbufF, hbufB,
             wsF, wrF, wsB, wrB, hsF, hrF, hsB, hrB):
        my = lax.axis_index("i")
        left = (my + N_DEV - 1) % N_DEV
        right = (my + 1) % N_DEV

        barrier = pltpu.get_barrier_semaphore()
        for nbr in (left, right):
            pl.semaphore_signal(barrier, inc=1, device_id=(nbr,),
                                device_id_type=pl.DeviceIdType.MESH)
        pl.semaphore_wait(barrier, 2)

        wbufF[0] = W_ref[...]
        wbufB[0] = W_ref[...]
        hbufF[0] = hist_ref[...]
        hbufB[0] = hist_ref[...]
        hists_ref[pl.ds(my, 1)] = hist_ref[...][None]
        out_ref[...] = jnp.zeros((n_tok, d_out), jnp.float32)

        x_v = x_ref[...]
        e_v = e_ref[...]
        rank_v = rank_ref[...]
        row = lax.broadcasted_iota(jnp.int32, (R, 1), 0)
        ej0 = row // L
        slotj = row % L

        def consume_block(src, wblk):
            sel = jnp.where((e_v == src * e_per + ej0) & (rank_v == slotj),
                            1.0, 0.0).astype(jnp.bfloat16)
            a = lax.dot_general(sel, x_v, (((1,), (0,)), ((), ())),
                                preferred_element_type=jnp.float32)
            a = a.astype(jnp.bfloat16)
            parts = []
            for k in range(e_per):
                r = lax.dot_general(a[k * L:(k + 1) * L], wblk[k],
                                    (((1,), (0,)), ((), ())),
                                    preferred_element_type=jnp.float32)
                parts.append(r.astype(jnp.bfloat16))
            r4 = jnp.concatenate(parts, axis=0)
            contrib = lax.dot_general(sel, r4, (((0,), (0,)), ((), ())),
                                      preferred_element_type=jnp.float32)
            out_ref[...] += contrib

        def hop(h, wbuf, hbuf, wsend, wrecv, hsend, hrecv, dst):
            ss, rs = h % 2, (h + 1) % 2
            wr_ = pltpu.make_async_remote_copy(
                src_ref=wbuf.at[ss], dst_ref=wbuf.at[rs],
                send_sem=wsend.at[ss], recv_sem=wrecv.at[rs],
                device_id=(dst,), device_id_type=pl.DeviceIdType.MESH)
            hr_ = pltpu.make_async_remote_copy(
                src_ref=hbuf.at[ss], dst_ref=hbuf.at[rs],
                send_sem=hsend.at[ss], recv_sem=hrecv.at[rs],
                device_id=(dst,), device_id_type=pl.DeviceIdType.MESH)
            wr_.start()
            hr_.start()
            return wr_, hr_

        fpend = hop(0, wbufF, hbufF, wsF, wrF, hsF, hrF, right)
        bpend = hop(0, wbufB, hbufB, wsB, wrB, hsB, hrB, left)

        consume_block(my, W_ref[...])

        for h in range(FWD):
            rs = (h + 1) % 2
            for r_ in fpend:
                r_.wait()
            if bpend is not None:
                for r_ in bpend:
                    r_.wait()
            fpend = (hop(h + 1, wbufF, hbufF, wsF, wrF, hsF, hrF, right)
                     if h + 1 < FWD else None)
            bpend = (hop(h + 1, wbufB, hbufB, wsB, wrB, hsB, hrB, left)
                     if h + 1 < BWD else None)
            src_f = (my + N_DEV - 1 - h) % N_DEV
            hists_ref[pl.ds(src_f, 1)] = hbufF[rs][None]
            consume_block(src_f, wbufF[rs])
            if h < BWD:
                src_b = (my + 1 + h) % N_DEV
                hists_ref[pl.ds(src_b, 1)] = hbufB[rs][None]
                consume_block(src_b, wbufB[rs])
            if fpend is None and bpend is not None:
                for r_ in bpend:
                    r_.wait()
                bpend = None

    out_k, hists = pl.pallas_call(
        body,
        out_shape=[
            jax.ShapeDtypeStruct((n_tok, d_out), jnp.float32),
            jax.ShapeDtypeStruct((N_DEV, 1, N_EXPERTS), jnp.int32),
        ],
        in_specs=[pl.BlockSpec(memory_space=pltpu.VMEM)] * 5,
        out_specs=[pl.BlockSpec(memory_space=pltpu.VMEM)] * 2,
        scratch_shapes=[
            pltpu.VMEM((2, e_per, d_model, d_out), jnp.bfloat16),
            pltpu.VMEM((2, e_per, d_model, d_out), jnp.bfloat16),
            pltpu.VMEM((2, 1, N_EXPERTS), jnp.int32),
            pltpu.VMEM((2, 1, N_EXPERTS), jnp.int32),
        ] + [pltpu.SemaphoreType.DMA((2,))] * 8,
        compiler_params=pltpu.CompilerParams(
            collective_id=0, vmem_limit_bytes=100 * 1024 * 1024),
    )(xb, W, hist[None, :], e[None, :], local_rank[None, :])

    hists = hists[:, 0, :]
    my = lax.axis_index("i")
    prior = (jnp.arange(N_DEV) < my).astype(jnp.int32)[:, None]
    gprefix = jnp.sum(hists * prior, axis=0)
    keep = gprefix[e] + local_rank < CAPACITY
    return jnp.where(keep[:, None], out_k, 0.0)


# device time: 449352 ns/iter; 1.8972x vs baseline; 1.1132x over previous
import jax
import jax.numpy as jnp
from jax import lax
from jax.experimental import pallas as pl
from jax.experimental.pallas import tpu as pltpu

N_DEV = 16
N_EXPERTS = 64
CAPACITY = 409
L = 128


def kernel(x, router_W, route_idx, expert_W):
    n_tok, d_model = x.shape
    e_per, _, d_out = expert_W.shape

    e = route_idx[:, 0]
    oh = jax.nn.one_hot(e, N_EXPERTS, dtype=jnp.int32)
    hist = jnp.sum(oh, axis=0, dtype=jnp.int32)
    local_rank = jnp.sum(oh * (jnp.cumsum(oh, axis=0) - oh), axis=1)

    xb = x.astype(jnp.bfloat16)
    W = expert_W.astype(jnp.bfloat16)

    FWD = N_DEV // 2
    BWD = N_DEV - 1 - FWD
    R = e_per * L

    def body(x_ref, W_ref, hist_ref, e_ref, ecol_ref, rank_ref, rankcol_ref,
             out_ref, hists_ref, wbufF, wbufB, hbufF, hbufB,
             wsF, wrF, wsB, wrB, hsF, hrF, hsB, hrB):
        my = lax.axis_index("i")
        left = (my + N_DEV - 1) % N_DEV
        right = (my + 1) % N_DEV

        barrier = pltpu.get_barrier_semaphore()
        for nbr in (left, right):
            pl.semaphore_signal(barrier, inc=1, device_id=(nbr,),
                                device_id_type=pl.DeviceIdType.MESH)
        pl.semaphore_wait(barrier, 2)

        wbufF[0] = W_ref[...]
        wbufB[0] = W_ref[...]
        hbufF[0] = hist_ref[...]
        hbufB[0] = hist_ref[...]
        hists_ref[pl.ds(my, 1)] = hist_ref[...][None]
        out_ref[...] = jnp.zeros((n_tok, d_out), jnp.float32)

        x_v = x_ref[...]
        e_v = e_ref[...]
        rank_v = rank_ref[...]
        row = lax.broadcasted_iota(jnp.int32, (R, 1), 0)
        ej0 = row // L
        slotj = row % L

        def consume_block(src, wblk):
            sel = jnp.where((e_v == src * e_per + ej0) & (rank_v == slotj),
                            1.0, 0.0).astype(jnp.bfloat16)
            a = lax.dot_general(sel, x_v, (((1,), (0,)), ((), ())),
                                preferred_element_type=jnp.float32)
            a = a.astype(jnp.bfloat16)
            parts = []
            for k in range(e_per):
                r = lax.dot_general(a[k * L:(k + 1) * L], wblk[k],
                                    (((1,), (0,)), ((), ())),
                                    preferred_element_type=jnp.float32)
                parts.append(r.astype(jnp.bfloat16))
            r4 = jnp.concatenate(parts, axis=0)
            contrib = lax.dot_general(sel, r4, (((0,), (0,)), ((), ())),
                                      preferred_element_type=jnp.float32)
            out_ref[...] += contrib

        def hop(h, wbuf, hbuf, wsend, wrecv, hsend, hrecv, dst):
            ss, rs = h % 2, (h + 1) % 2
            wr_ = pltpu.make_async_remote_copy(
                src_ref=wbuf.at[ss], dst_ref=wbuf.at[rs],
                send_sem=wsend.at[ss], recv_sem=wrecv.at[rs],
                device_id=(dst,), device_id_type=pl.DeviceIdType.MESH)
            hr_ = pltpu.make_async_remote_copy(
                src_ref=hbuf.at[ss], dst_ref=hbuf.at[rs],
                send_sem=hsend.at[ss], recv_sem=hrecv.at[rs],
                device_id=(dst,), device_id_type=pl.DeviceIdType.MESH)
            wr_.start()
            hr_.start()
            return wr_, hr_

        fpend = hop(0, wbufF, hbufF, wsF, wrF, hsF, hrF, right)
        bpend = hop(0, wbufB, hbufB, wsB, wrB, hsB, hrB, left)

        consume_block(my, W_ref[...])

        for h in range(FWD):
            rs = (h + 1) % 2
            for r_ in fpend:
                r_.wait()
            if bpend is not None:
                for r_ in bpend:
                    r_.wait()
            fpend = (hop(h + 1, wbufF, hbufF, wsF, wrF, hsF, hrF, right)
                     if h + 1 < FWD else None)
            bpend = (hop(h + 1, wbufB, hbufB, wsB, wrB, hsB, hrB, left)
                     if h + 1 < BWD else None)
            src_f = (my + N_DEV - 1 - h) % N_DEV
            hists_ref[pl.ds(src_f, 1)] = hbufF[rs][None]
            consume_block(src_f, wbufF[rs])
            if h < BWD:
                src_b = (my + 1 + h) % N_DEV
                hists_ref[pl.ds(src_b, 1)] = hbufB[rs][None]
                consume_block(src_b, wbufB[rs])
        hall = hists_ref[...]
        dmask = (lax.broadcasted_iota(jnp.int32, (N_DEV, 1, N_EXPERTS), 0)
                 < my).astype(jnp.int32)
        gpref = jnp.sum(hall * dmask, axis=0).astype(jnp.float32)
        ohcol = (ecol_ref[...] == lax.broadcasted_iota(
            jnp.int32, (n_tok, N_EXPERTS), 1)).astype(jnp.float32)
        gpref_tok = jnp.sum(ohcol * gpref, axis=1, keepdims=True)
        keep = (gpref_tok + rankcol_ref[...].astype(jnp.float32)
                < float(CAPACITY)).astype(jnp.float32)
        out_ref[...] *= keep

    out_k = pl.pallas_call(
        body,
        out_shape=jax.ShapeDtypeStruct((n_tok, d_out), jnp.float32),
        in_specs=[pl.BlockSpec(memory_space=pltpu.VMEM)] * 7,
        out_specs=pl.BlockSpec(memory_space=pltpu.VMEM),
        scratch_shapes=[
            pltpu.VMEM((N_DEV, 1, N_EXPERTS), jnp.int32),
            pltpu.VMEM((2, e_per, d_model, d_out), jnp.bfloat16),
            pltpu.VMEM((2, e_per, d_model, d_out), jnp.bfloat16),
            pltpu.VMEM((2, 1, N_EXPERTS), jnp.int32),
            pltpu.VMEM((2, 1, N_EXPERTS), jnp.int32),
        ] + [pltpu.SemaphoreType.DMA((2,))] * 8,
        compiler_params=pltpu.CompilerParams(
            collective_id=0, vmem_limit_bytes=100 * 1024 * 1024),
    )(xb, W, hist[None, :], e[None, :], route_idx,
      local_rank[None, :], local_rank[:, None])

    return out_k


# device time: 200238 ns/iter; 4.2575x vs baseline; 2.2441x over previous
import jax
import jax.numpy as jnp
from jax import lax
from jax.experimental import pallas as pl
from jax.experimental.pallas import tpu as pltpu

N_DEV = 16
N_EXPERTS = 64
CAPACITY = 409
L2 = 64


def kernel(x, router_W, route_idx, expert_W):
    n_tok, d_model = x.shape
    e_per, _, d_out = expert_W.shape
    R2 = e_per * L2
    TOT = N_DEV * R2

    e = route_idx[:, 0]
    oh = jax.nn.one_hot(e, N_EXPERTS, dtype=jnp.int32)
    hist = jnp.sum(oh, axis=0, dtype=jnp.int32)
    local_rank = jnp.sum(oh * (jnp.cumsum(oh, axis=0) - oh), axis=1)

    xb = x.astype(jnp.bfloat16)
    W = expert_W.astype(jnp.bfloat16)

    MESH = pl.DeviceIdType.MESH

    G = 4
    NG = N_DEV // G

    def body(x_ref, W_ref, hist_ref, e_ref, rank_ref, out_ref,
             S_ref, R_ref, O_ref, OB_ref, HR_ref,
             dsend, drecv, hsend, hrecv, csend, crecv):
        my = lax.axis_index("i")

        barrier = pltpu.get_barrier_semaphore()
        for p in range(N_DEV):
            @pl.when(p != my)
            def _():
                pl.semaphore_signal(barrier, inc=1, device_id=(p,),
                                    device_id_type=MESH)
        pl.semaphore_wait(barrier, N_DEV - 1)

        def selg(g):
            j0 = lax.broadcasted_iota(jnp.int32, (G * R2, 1), 0) + g * G * R2
            return jnp.where(
                (e_ref[...] == j0 // L2) & (rank_ref[...] == j0 % L2),
                1.0, 0.0).astype(jnp.bfloat16)

        for g in range(NG):
            packed = lax.dot_general(selg(g), x_ref[...],
                                     (((1,), (0,)), ((), ())),
                                     preferred_element_type=jnp.float32)
            S_ref[g * G:(g + 1) * G] = jnp.reshape(
                packed.astype(jnp.bfloat16), (G, R2, d_model))

        d_rdmas = []
        for d in range(N_DEV):
            dr = pltpu.make_async_remote_copy(
                src_ref=S_ref.at[d], dst_ref=R_ref.at[my],
                send_sem=dsend.at[d], recv_sem=drecv.at[my],
                device_id=(d,), device_id_type=MESH)
            hr = pltpu.make_async_remote_copy(
                src_ref=hist_ref, dst_ref=HR_ref.at[my],
                send_sem=hsend.at[d], recv_sem=hrecv.at[my],
                device_id=(d,), device_id_type=MESH)
            d_rdmas.append((dr, hr))

            @pl.when(p_eq(d, my))
            def _():
                R_ref[d] = S_ref[d]
                HR_ref[d] = hist_ref[...]

            @pl.when(jnp.logical_not(p_eq(d, my)))
            def _(dr=dr, hr=hr):
                dr.start()
                hr.start()

        for s in range(N_DEV):
            rwait = pltpu.make_async_remote_copy(
                src_ref=S_ref.at[s], dst_ref=R_ref.at[s],
                send_sem=dsend.at[s], recv_sem=drecv.at[s],
                device_id=(s,), device_id_type=MESH)
            hwait = pltpu.make_async_remote_copy(
                src_ref=hist_ref, dst_ref=HR_ref.at[s],
                send_sem=hsend.at[s], recv_sem=hrecv.at[s],
                device_id=(s,), device_id_type=MESH)

            @pl.when(jnp.logical_not(p_eq(s, my)))
            def _(rwait=rwait, hwait=hwait):
                rwait.wait_recv()
                hwait.wait_recv()

        HRv = HR_ref[...][:, 0, :].astype(jnp.float32)
        lt = (lax.broadcasted_iota(jnp.int32, (N_DEV, N_DEV), 0)
              > lax.broadcasted_iota(jnp.int32, (N_DEV, N_DEV), 1)
              ).astype(jnp.float32)
        rowj = lax.broadcasted_iota(jnp.int32, (N_DEV * L2, 1), 0)
        src_oh = (rowj // L2 == lax.broadcasted_iota(
            jnp.int32, (N_DEV * L2, N_DEV), 1)).astype(jnp.float32)
        slot_col = (rowj % L2).astype(jnp.float32)
        for k in range(e_per):
            ohk = (lax.broadcasted_iota(jnp.int32, (1, N_EXPERTS), 1)
                   == my * e_per + k).astype(jnp.float32)
            cnts = jnp.sum(HRv * ohk, axis=1, keepdims=True)
            prefix = lax.dot_general(lt, cnts, (((1,), (0,)), ((), ())),
                                     preferred_element_type=jnp.float32)
            prefix_col = lax.dot_general(
                src_oh, prefix, (((1,), (0,)), ((), ())),
                precision=lax.Precision.HIGHEST,
                preferred_element_type=jnp.float32)
            keep_col = ((prefix_col + slot_col) < float(CAPACITY)
                        ).astype(jnp.float32)
            Ak = jnp.reshape(R_ref[...][:, k * L2:(k + 1) * L2, :],
                             (N_DEV * L2, d_model))
            Yk = lax.dot_general(Ak, W_ref[k], (((1,), (0,)), ((), ())),
                                 preferred_element_type=jnp.float32)
            Yk = Yk * keep_col
            O_ref[:, k * L2:(k + 1) * L2, :] = jnp.reshape(
                Yk.astype(jnp.bfloat16), (N_DEV, L2, d_out))

        c_rdmas = []
        for d in range(N_DEV):
            cr = pltpu.make_async_remote_copy(
                src_ref=O_ref.at[d], dst_ref=OB_ref.at[my],
                send_sem=csend.at[d], recv_sem=crecv.at[my],
                device_id=(d,), device_id_type=MESH)
            c_rdmas.append(cr)

            @pl.when(p_eq(d, my))
            def _():
                OB_ref[d] = O_ref[d]

            @pl.when(jnp.logical_not(p_eq(d, my)))
            def _(cr=cr):
                cr.start()

        for s in range(N_DEV):
            cwait = pltpu.make_async_remote_copy(
                src_ref=O_ref.at[s], dst_ref=OB_ref.at[s],
                send_sem=csend.at[s], recv_sem=crecv.at[s],
                device_id=(s,), device_id_type=MESH)

            @pl.when(jnp.logical_not(p_eq(s, my)))
            def _(cwait=cwait):
                cwait.wait_recv()

        out_ref[...] = jnp.zeros((n_tok, d_out), jnp.float32)
        for g in range(NG):
            OBg = jnp.reshape(OB_ref[...][g * G:(g + 1) * G], (G * R2, d_out))
            out_ref[...] += lax.dot_general(
                selg(g), OBg, (((0,), (0,)), ((), ())),
                preferred_element_type=jnp.float32)

        for d in range(N_DEV):
            dr, hr = d_rdmas[d]
            cr = c_rdmas[d]

            @pl.when(jnp.logical_not(p_eq(d, my)))
            def _(dr=dr, hr=hr, cr=cr):
                dr.wait_send()
                hr.wait_send()
                cr.wait_send()

    def p_eq(a, b):
        return jnp.equal(a, b)

    out_k = pl.pallas_call(
        body,
        out_shape=jax.ShapeDtypeStruct((n_tok, d_out), jnp.float32),
        in_specs=[pl.BlockSpec(memory_space=pltpu.VMEM)] * 5,
        out_specs=pl.BlockSpec(memory_space=pltpu.VMEM),
        scratch_shapes=[
            pltpu.VMEM((N_DEV, R2, d_model), jnp.bfloat16),
            pltpu.VMEM((N_DEV, R2, d_model), jnp.bfloat16),
            pltpu.VMEM((N_DEV, R2, d_out), jnp.bfloat16),
            pltpu.VMEM((N_DEV, R2, d_out), jnp.bfloat16),
            pltpu.VMEM((N_DEV, 1, N_EXPERTS), jnp.int32),
        ] + [pltpu.SemaphoreType.DMA((N_DEV,))] * 6,
        compiler_params=pltpu.CompilerParams(
            collective_id=0, vmem_limit_bytes=100 * 1024 * 1024),
    )(xb, W, hist[None, :], e[None, :], local_rank[None, :])

    return out_k


# device time: 183744 ns/iter; 4.6397x vs baseline; 1.0898x over previous
import jax
import jax.numpy as jnp
from jax import lax
from jax.experimental import pallas as pl
from jax.experimental.pallas import tpu as pltpu

N_DEV = 16
N_EXPERTS = 64
CAPACITY = 409
L2 = 64


def kernel(x, router_W, route_idx, expert_W):
    n_tok, d_model = x.shape
    e_per, _, d_out = expert_W.shape
    R2 = e_per * L2
    TOT = N_DEV * R2

    e = route_idx[:, 0]
    oh = jax.nn.one_hot(e, N_EXPERTS, dtype=jnp.int32)
    hist = jnp.sum(oh, axis=0, dtype=jnp.int32)
    local_rank = jnp.sum(oh * (jnp.cumsum(oh, axis=0) - oh), axis=1)

    xb = x.astype(jnp.bfloat16)
    W = expert_W.astype(jnp.bfloat16)

    MESH = pl.DeviceIdType.MESH

    G = 4
    NG = N_DEV // G

    def body(x_ref, W_ref, hist_ref, e_ref, rank_ref, out_ref,
             S_ref, R_ref, O_ref, OB_ref, HR_ref,
             dsend, drecv, hsend, hrecv, csend, crecv):
        my = lax.axis_index("i")

        barrier = pltpu.get_barrier_semaphore()
        for p in range(N_DEV):
            @pl.when(p != my)
            def _():
                pl.semaphore_signal(barrier, inc=1, device_id=(p,),
                                    device_id_type=MESH)
        pl.semaphore_wait(barrier, N_DEV - 1)

        d_rdmas = []
        for d in range(N_DEV):
            hr = pltpu.make_async_remote_copy(
                src_ref=hist_ref, dst_ref=HR_ref.at[my],
                send_sem=hsend.at[d], recv_sem=hrecv.at[my],
                device_id=(d,), device_id_type=MESH)
            d_rdmas.append(hr)

            @pl.when(p_eq(d, my))
            def _():
                HR_ref[d] = hist_ref[...]

            @pl.when(jnp.logical_not(p_eq(d, my)))
            def _(hr=hr):
                hr.start()

        def selg(g):
            j0 = lax.broadcasted_iota(jnp.int32, (G * R2, 1), 0) + g * G * R2
            return jnp.where(
                (e_ref[...] == j0 // L2) & (rank_ref[...] == j0 % L2),
                1.0, 0.0).astype(jnp.bfloat16)

        for g in range(NG):
            packed = lax.dot_general(selg(g), x_ref[...],
                                     (((1,), (0,)), ((), ())),
                                     preferred_element_type=jnp.float32)
            S_ref[g * G:(g + 1) * G] = jnp.reshape(
                packed.astype(jnp.bfloat16), (G, R2, d_model))
            for i in range(G):
                d = g * G + i
                dr = pltpu.make_async_remote_copy(
                    src_ref=S_ref.at[d], dst_ref=R_ref.at[my],
                    send_sem=dsend.at[d], recv_sem=drecv.at[my],
                    device_id=(d,), device_id_type=MESH)
                d_rdmas.append(dr)

                @pl.when(p_eq(d, my))
                def _(d=d):
                    R_ref[d] = S_ref[d]

                @pl.when(jnp.logical_not(p_eq(d, my)))
                def _(dr=dr):
                    dr.start()

        for s in range(N_DEV):
            rwait = pltpu.make_async_remote_copy(
                src_ref=S_ref.at[s], dst_ref=R_ref.at[s],
                send_sem=dsend.at[s], recv_sem=drecv.at[s],
                device_id=(s,), device_id_type=MESH)
            hwait = pltpu.make_async_remote_copy(
                src_ref=hist_ref, dst_ref=HR_ref.at[s],
                send_sem=hsend.at[s], recv_sem=hrecv.at[s],
                device_id=(s,), device_id_type=MESH)

            @pl.when(jnp.logical_not(p_eq(s, my)))
            def _(rwait=rwait, hwait=hwait):
                rwait.wait_recv()
                hwait.wait_recv()

        HRv = HR_ref[...][:, 0, :].astype(jnp.float32)
        lt = (lax.broadcasted_iota(jnp.int32, (N_DEV, N_DEV), 0)
              > lax.broadcasted_iota(jnp.int32, (N_DEV, N_DEV), 1)
              ).astype(jnp.float32)
        rowj = lax.broadcasted_iota(jnp.int32, (N_DEV * L2, 1), 0)
        src_oh = (rowj // L2 == lax.broadcasted_iota(
            jnp.int32, (N_DEV * L2, N_DEV), 1)).astype(jnp.float32)
        slot_col = (rowj % L2).astype(jnp.float32)
        for k in range(e_per):
            ohk = (lax.broadcasted_iota(jnp.int32, (1, N_EXPERTS), 1)
                   == my * e_per + k).astype(jnp.float32)
            cnts = jnp.sum(HRv * ohk, axis=1, keepdims=True)
            prefix = lax.dot_general(lt, cnts, (((1,), (0,)), ((), ())),
                                     preferred_element_type=jnp.float32)
            prefix_col = lax.dot_general(
                src_oh, prefix, (((1,), (0,)), ((), ())),
                precision=lax.Precision.HIGHEST,
                preferred_element_type=jnp.float32)
            keep_col = ((prefix_col + slot_col) < float(CAPACITY)
                        ).astype(jnp.float32)
            Ak = jnp.reshape(R_ref[...][:, k * L2:(k + 1) * L2, :],
                             (N_DEV * L2, d_model))
            Yk = lax.dot_general(Ak, W_ref[k], (((1,), (0,)), ((), ())),
                                 preferred_element_type=jnp.float32)
            Yk = Yk * keep_col
            O_ref[:, k * L2:(k + 1) * L2, :] = jnp.reshape(
                Yk.astype(jnp.bfloat16), (N_DEV, L2, d_out))

        c_rdmas = []
        for d in range(N_DEV):
            cr = pltpu.make_async_remote_copy(
                src_ref=O_ref.at[d], dst_ref=OB_ref.at[my],
                send_sem=csend.at[d], recv_sem=crecv.at[my],
                device_id=(d,), device_id_type=MESH)
            c_rdmas.append(cr)

            @pl.when(p_eq(d, my))
            def _():
                OB_ref[d] = O_ref[d]

            @pl.when(jnp.logical_not(p_eq(d, my)))
            def _(cr=cr):
                cr.start()

        out_ref[...] = jnp.zeros((n_tok, d_out), jnp.float32)
        for g in range(NG):
            for i in range(G):
                s = g * G + i
                cwait = pltpu.make_async_remote_copy(
                    src_ref=O_ref.at[s], dst_ref=OB_ref.at[s],
                    send_sem=csend.at[s], recv_sem=crecv.at[s],
                    device_id=(s,), device_id_type=MESH)

                @pl.when(jnp.logical_not(p_eq(s, my)))
                def _(cwait=cwait):
                    cwait.wait_recv()

            OBg = jnp.reshape(OB_ref[...][g * G:(g + 1) * G], (G * R2, d_out))
            out_ref[...] += lax.dot_general(
                selg(g), OBg, (((0,), (0,)), ((), ())),
                preferred_element_type=jnp.float32)

        for d in range(N_DEV):
            hr = d_rdmas[d]
            dr = d_rdmas[N_DEV + d]
            cr = c_rdmas[d]

            @pl.when(jnp.logical_not(p_eq(d, my)))
            def _(dr=dr, hr=hr, cr=cr):
                dr.wait_send()
                hr.wait_send()
                cr.wait_send()

    def p_eq(a, b):
        return jnp.equal(a, b)

    out_k = pl.pallas_call(
        body,
        out_shape=jax.ShapeDtypeStruct((n_tok, d_out), jnp.float32),
        in_specs=[pl.BlockSpec(memory_space=pltpu.VMEM)] * 5,
        out_specs=pl.BlockSpec(memory_space=pltpu.VMEM),
        scratch_shapes=[
            pltpu.VMEM((N_DEV, R2, d_model), jnp.bfloat16),
            pltpu.VMEM((N_DEV, R2, d_model), jnp.bfloat16),
            pltpu.VMEM((N_DEV, R2, d_out), jnp.bfloat16),
            pltpu.VMEM((N_DEV, R2, d_out), jnp.bfloat16),
            pltpu.VMEM((N_DEV, 1, N_EXPERTS), jnp.int32),
        ] + [pltpu.SemaphoreType.DMA((N_DEV,))] * 6,
        compiler_params=pltpu.CompilerParams(
            collective_id=0, vmem_limit_bytes=100 * 1024 * 1024),
    )(xb, W, hist[None, :], e[None, :], local_rank[None, :])

    return out_k


# device time: 182015 ns/iter; 4.6838x vs baseline; 1.0095x over previous
import jax
import jax.numpy as jnp
from jax import lax
from jax.experimental import pallas as pl
from jax.experimental.pallas import tpu as pltpu

N_DEV = 16
N_EXPERTS = 64
CAPACITY = 409
L2 = 64


def kernel(x, router_W, route_idx, expert_W):
    n_tok, d_model = x.shape
    e_per, _, d_out = expert_W.shape
    R2 = e_per * L2
    TOT = N_DEV * R2

    e = route_idx[:, 0]
    oh = jax.nn.one_hot(e, N_EXPERTS, dtype=jnp.int32)
    hist = jnp.sum(oh, axis=0, dtype=jnp.int32)
    local_rank = jnp.sum(oh * (jnp.cumsum(oh, axis=0) - oh), axis=1)

    xb = x.astype(jnp.bfloat16)
    W = expert_W.astype(jnp.bfloat16)

    MESH = pl.DeviceIdType.MESH

    G = 4
    NG = N_DEV // G

    def body(x_ref, W_ref, hist_ref, e_ref, rank_ref, out_ref,
             S_ref, R_ref, O_ref, OB_ref, HR_ref,
             dsend, drecv, hsend, hrecv, csend, crecv):
        my = lax.axis_index("i")

        barrier = pltpu.get_barrier_semaphore()
        for p in range(N_DEV):
            @pl.when(p != my)
            def _():
                pl.semaphore_signal(barrier, inc=1, device_id=(p,),
                                    device_id_type=MESH)
        pl.semaphore_wait(barrier, N_DEV - 1)

        d_rdmas = []
        for d in range(N_DEV):
            hr = pltpu.make_async_remote_copy(
                src_ref=hist_ref, dst_ref=HR_ref.at[my],
                send_sem=hsend.at[d], recv_sem=hrecv.at[my],
                device_id=(d,), device_id_type=MESH)
            d_rdmas.append(hr)

            @pl.when(p_eq(d, my))
            def _():
                HR_ref[d] = hist_ref[...]

            @pl.when(jnp.logical_not(p_eq(d, my)))
            def _(hr=hr):
                hr.start()

        def selg(g):
            j0 = lax.broadcasted_iota(jnp.int32, (G * R2, 1), 0) + g * G * R2
            return jnp.where(
                (e_ref[...] == j0 // L2) & (rank_ref[...] == j0 % L2),
                1.0, 0.0).astype(jnp.bfloat16)

        for g in range(NG):
            packed = lax.dot_general(selg(g), x_ref[...],
                                     (((1,), (0,)), ((), ())),
                                     preferred_element_type=jnp.float32)
            S_ref[g * G:(g + 1) * G] = jnp.reshape(
                packed.astype(jnp.bfloat16), (G, R2, d_model))
            for i in range(G):
                d = g * G + i
                dr = pltpu.make_async_remote_copy(
                    src_ref=S_ref.at[d], dst_ref=R_ref.at[my],
                    send_sem=dsend.at[d], recv_sem=drecv.at[my],
                    device_id=(d,), device_id_type=MESH)
                d_rdmas.append(dr)

                @pl.when(p_eq(d, my))
                def _(d=d):
                    R_ref[d] = S_ref[d]

                @pl.when(jnp.logical_not(p_eq(d, my)))
                def _(dr=dr):
                    dr.start()

        for s in range(N_DEV):
            hwait = pltpu.make_async_remote_copy(
                src_ref=hist_ref, dst_ref=HR_ref.at[s],
                send_sem=hsend.at[s], recv_sem=hrecv.at[s],
                device_id=(s,), device_id_type=MESH)

            @pl.when(jnp.logical_not(p_eq(s, my)))
            def _(hwait=hwait):
                hwait.wait_recv()

        HRv = HR_ref[...][:, 0, :].astype(jnp.float32)
        lt = (lax.broadcasted_iota(jnp.int32, (N_DEV, N_DEV), 0)
              > lax.broadcasted_iota(jnp.int32, (N_DEV, N_DEV), 1)
              ).astype(jnp.float32)
        m4 = (lax.broadcasted_iota(jnp.int32, (N_EXPERTS, e_per), 0)
              == my * e_per + lax.broadcasted_iota(
                  jnp.int32, (N_EXPERTS, e_per), 1)).astype(jnp.float32)
        cnts4 = lax.dot_general(HRv, m4, (((1,), (0,)), ((), ())),
                                precision=lax.Precision.HIGHEST,
                                preferred_element_type=jnp.float32)
        p4 = lax.dot_general(lt, cnts4, (((1,), (0,)), ((), ())),
                             precision=lax.Precision.HIGHEST,
                             preferred_element_type=jnp.float32)
        slot64 = lax.broadcasted_iota(
            jnp.int32, (L2, 1), 0).astype(jnp.float32)

        c_rdmas = []
        for s in range(N_DEV):
            rwait = pltpu.make_async_remote_copy(
                src_ref=S_ref.at[s], dst_ref=R_ref.at[s],
                send_sem=dsend.at[s], recv_sem=drecv.at[s],
                device_id=(s,), device_id_type=MESH)

            @pl.when(jnp.logical_not(p_eq(s, my)))
            def _(rwait=rwait):
                rwait.wait_recv()

            Rs = R_ref[s]
            for k in range(e_per):
                keep = ((p4[s, k] + slot64) < float(CAPACITY)
                        ).astype(jnp.float32)
                Yk = lax.dot_general(Rs[k * L2:(k + 1) * L2], W_ref[k],
                                     (((1,), (0,)), ((), ())),
                                     preferred_element_type=jnp.float32)
                O_ref[s, k * L2:(k + 1) * L2, :] = (
                    Yk * keep).astype(jnp.bfloat16)

            cr = pltpu.make_async_remote_copy(
                src_ref=O_ref.at[s], dst_ref=OB_ref.at[my],
                send_sem=csend.at[s], recv_sem=crecv.at[my],
                device_id=(s,), device_id_type=MESH)
            c_rdmas.append(cr)

            @pl.when(p_eq(s, my))
            def _(s=s):
                OB_ref[s] = O_ref[s]

            @pl.when(jnp.logical_not(p_eq(s, my)))
            def _(cr=cr):
                cr.start()

        out_ref[...] = jnp.zeros((n_tok, d_out), jnp.float32)
        for g in range(NG):
            for i in range(G):
                s = g * G + i
                cwait = pltpu.make_async_remote_copy(
                    src_ref=O_ref.at[s], dst_ref=OB_ref.at[s],
                    send_sem=csend.at[s], recv_sem=crecv.at[s],
                    device_id=(s,), device_id_type=MESH)

                @pl.when(jnp.logical_not(p_eq(s, my)))
                def _(cwait=cwait):
                    cwait.wait_recv()

            OBg = jnp.reshape(OB_ref[...][g * G:(g + 1) * G], (G * R2, d_out))
            out_ref[...] += lax.dot_general(
                selg(g), OBg, (((0,), (0,)), ((), ())),
                preferred_element_type=jnp.float32)

        for d in range(N_DEV):
            hr = d_rdmas[d]
            dr = d_rdmas[N_DEV + d]
            cr = c_rdmas[d]

            @pl.when(jnp.logical_not(p_eq(d, my)))
            def _(dr=dr, hr=hr, cr=cr):
                dr.wait_send()
                hr.wait_send()
                cr.wait_send()

    def p_eq(a, b):
        return jnp.equal(a, b)

    out_k = pl.pallas_call(
        body,
        out_shape=jax.ShapeDtypeStruct((n_tok, d_out), jnp.float32),
        in_specs=[pl.BlockSpec(memory_space=pltpu.VMEM)] * 5,
        out_specs=pl.BlockSpec(memory_space=pltpu.VMEM),
        scratch_shapes=[
            pltpu.VMEM((N_DEV, R2, d_model), jnp.bfloat16),
            pltpu.VMEM((N_DEV, R2, d_model), jnp.bfloat16),
            pltpu.VMEM((N_DEV, R2, d_out), jnp.bfloat16),
            pltpu.VMEM((N_DEV, R2, d_out), jnp.bfloat16),
            pltpu.VMEM((N_DEV, 1, N_EXPERTS), jnp.int32),
        ] + [pltpu.SemaphoreType.DMA((N_DEV,))] * 6,
        compiler_params=pltpu.CompilerParams(
            collective_id=0, vmem_limit_bytes=100 * 1024 * 1024),
    )(xb, W, hist[None, :], e[None, :], local_rank[None, :])

    return out_k


# device time: 167410 ns/iter; 5.0924x vs baseline; 1.0872x over previous
import jax
import jax.numpy as jnp
from jax import lax
from jax.experimental import pallas as pl
from jax.experimental.pallas import tpu as pltpu

N_DEV = 16
N_EXPERTS = 64
CAPACITY = 409
L2 = 64


def kernel(x, router_W, route_idx, expert_W):
    n_tok, d_model = x.shape
    e_per, _, d_out = expert_W.shape
    R2 = e_per * L2
    TOT = N_DEV * R2

    e = route_idx[:, 0]
    oh = jax.nn.one_hot(e, N_EXPERTS, dtype=jnp.int32)
    hist = jnp.sum(oh, axis=0, dtype=jnp.int32)
    idx = jnp.arange(n_tok, dtype=jnp.int32)
    local_rank = jnp.sum(
        ((e[None, :] == e[:, None]) & (idx[:, None] < idx[None, :])
         ).astype(jnp.int32), axis=0)

    xb = x.astype(jnp.bfloat16)
    W = expert_W.astype(jnp.bfloat16)

    MESH = pl.DeviceIdType.MESH

    G = 4
    NG = N_DEV // G

    def body(x_ref, W_ref, hist_ref, e_ref, rank_ref, out_ref,
             S_ref, R_ref, O_ref, OB_ref, HR_ref,
             dsend, drecv, hsend, hrecv, csend, crecv):
        my = lax.axis_index("i")

        barrier = pltpu.get_barrier_semaphore()
        for p in range(N_DEV):
            @pl.when(p != my)
            def _():
                pl.semaphore_signal(barrier, inc=1, device_id=(p,),
                                    device_id_type=MESH)
        pl.semaphore_wait(barrier, N_DEV - 1)

        d_rdmas = []
        for d in range(N_DEV):
            hr = pltpu.make_async_remote_copy(
                src_ref=hist_ref, dst_ref=HR_ref.at[my],
                send_sem=hsend.at[d], recv_sem=hrecv.at[my],
                device_id=(d,), device_id_type=MESH)
            d_rdmas.append(hr)

            @pl.when(p_eq(d, my))
            def _():
                HR_ref[d] = hist_ref[...]

            @pl.when(jnp.logical_not(p_eq(d, my)))
            def _(hr=hr):
                hr.start()

        def selg(g):
            j0 = lax.broadcasted_iota(jnp.int32, (G * R2, 1), 0) + g * G * R2
            return jnp.where(
                (e_ref[...] == j0 // L2) & (rank_ref[...] == j0 % L2),
                1.0, 0.0).astype(jnp.bfloat16)

        for g in range(NG):
            packed = lax.dot_general(selg(g), x_ref[...],
                                     (((1,), (0,)), ((), ())),
                                     preferred_element_type=jnp.float32)
            S_ref[g * G:(g + 1) * G] = jnp.reshape(
                packed.astype(jnp.bfloat16), (G, R2, d_model))
            for i in range(G):
                d = g * G + i
                dr = pltpu.make_async_remote_copy(
                    src_ref=S_ref.at[d], dst_ref=R_ref.at[my],
                    send_sem=dsend.at[d], recv_sem=drecv.at[my],
                    device_id=(d,), device_id_type=MESH)
                d_rdmas.append(dr)

                @pl.when(p_eq(d, my))
                def _(d=d):
                    R_ref[d] = S_ref[d]

                @pl.when(jnp.logical_not(p_eq(d, my)))
                def _(dr=dr):
                    dr.start()

        for s in range(N_DEV):
            hwait = pltpu.make_async_remote_copy(
                src_ref=hist_ref, dst_ref=HR_ref.at[s],
                send_sem=hsend.at[s], recv_sem=hrecv.at[s],
                device_id=(s,), device_id_type=MESH)

            @pl.when(jnp.logical_not(p_eq(s, my)))
            def _(hwait=hwait):
                hwait.wait_recv()

        HRv = HR_ref[...][:, 0, :].astype(jnp.float32)
        lt = (lax.broadcasted_iota(jnp.int32, (N_DEV, N_DEV), 0)
              > lax.broadcasted_iota(jnp.int32, (N_DEV, N_DEV), 1)
              ).astype(jnp.float32)
        m4 = (lax.broadcasted_iota(jnp.int32, (N_EXPERTS, e_per), 0)
              == my * e_per + lax.broadcasted_iota(
                  jnp.int32, (N_EXPERTS, e_per), 1)).astype(jnp.float32)
        cnts4 = lax.dot_general(HRv, m4, (((1,), (0,)), ((), ())),
                                precision=lax.Precision.HIGHEST,
                                preferred_element_type=jnp.float32)
        p4 = lax.dot_general(lt, cnts4, (((1,), (0,)), ((), ())),
                             precision=lax.Precision.HIGHEST,
                             preferred_element_type=jnp.float32)
        slot64 = lax.broadcasted_iota(
            jnp.int32, (L2, 1), 0).astype(jnp.float32)

        c_rdmas = []
        for s in range(N_DEV):
            rwait = pltpu.make_async_remote_copy(
                src_ref=S_ref.at[s], dst_ref=R_ref.at[s],
                send_sem=dsend.at[s], recv_sem=drecv.at[s],
                device_id=(s,), device_id_type=MESH)

            @pl.when(jnp.logical_not(p_eq(s, my)))
            def _(rwait=rwait):
                rwait.wait_recv()

            Rs = R_ref[s]
            for k in range(e_per):
                keep = ((p4[s, k] + slot64) < float(CAPACITY)
                        ).astype(jnp.float32)
                Yk = lax.dot_general(Rs[k * L2:(k + 1) * L2], W_ref[k],
                                     (((1,), (0,)), ((), ())),
                                     preferred_element_type=jnp.float32)
                O_ref[s, k * L2:(k + 1) * L2, :] = (
                    Yk * keep).astype(jnp.bfloat16)

            cr = pltpu.make_async_remote_copy(
                src_ref=O_ref.at[s], dst_ref=OB_ref.at[my],
                send_sem=csend.at[s], recv_sem=crecv.at[my],
                device_id=(s,), device_id_type=MESH)
            c_rdmas.append(cr)

            @pl.when(p_eq(s, my))
            def _(s=s):
                OB_ref[s] = O_ref[s]

            @pl.when(jnp.logical_not(p_eq(s, my)))
            def _(cr=cr):
                cr.start()

        out_ref[...] = jnp.zeros((n_tok, d_out), jnp.float32)
        for g in range(NG):
            for i in range(G):
                s = g * G + i
                cwait = pltpu.make_async_remote_copy(
                    src_ref=O_ref.at[s], dst_ref=OB_ref.at[s],
                    send_sem=csend.at[s], recv_sem=crecv.at[s],
                    device_id=(s,), device_id_type=MESH)

                @pl.when(jnp.logical_not(p_eq(s, my)))
                def _(cwait=cwait):
                    cwait.wait_recv()

            OBg = jnp.reshape(OB_ref[...][g * G:(g + 1) * G], (G * R2, d_out))
            out_ref[...] += lax.dot_general(
                selg(g), OBg, (((0,), (0,)), ((), ())),
                preferred_element_type=jnp.float32)

        for d in range(N_DEV):
            hr = d_rdmas[d]
            dr = d_rdmas[N_DEV + d]
            cr = c_rdmas[d]

            @pl.when(jnp.logical_not(p_eq(d, my)))
            def _(dr=dr, hr=hr, cr=cr):
                dr.wait_send()
                hr.wait_send()
                cr.wait_send()

    def p_eq(a, b):
        return jnp.equal(a, b)

    out_k = pl.pallas_call(
        body,
        out_shape=jax.ShapeDtypeStruct((n_tok, d_out), jnp.float32),
        in_specs=[pl.BlockSpec(memory_space=pltpu.VMEM)] * 5,
        out_specs=pl.BlockSpec(memory_space=pltpu.VMEM),
        scratch_shapes=[
            pltpu.VMEM((N_DEV, R2, d_model), jnp.bfloat16),
            pltpu.VMEM((N_DEV, R2, d_model), jnp.bfloat16),
            pltpu.VMEM((N_DEV, R2, d_out), jnp.bfloat16),
            pltpu.VMEM((N_DEV, R2, d_out), jnp.bfloat16),
            pltpu.VMEM((N_DEV, 1, N_EXPERTS), jnp.int32),
        ] + [pltpu.SemaphoreType.DMA((N_DEV,))] * 6,
        compiler_params=pltpu.CompilerParams(
            collective_id=0, vmem_limit_bytes=100 * 1024 * 1024),
    )(xb, W, hist[None, :], e[None, :], local_rank[None, :])

    return out_k
